# baseline (speedup 1.0000x reference)
import jax
import jax.numpy as jnp
from jax import lax
from jax.experimental import pallas as pl
from jax.experimental.pallas import tpu as pltpu
from jax.experimental.pallas import tpu_sc as plsc

_B = 256
_CROP = 224
_C = 3
_H = 4096
_W = 4096
_NC = 2
_NS = 16
_NW = _NC * _NS
_PER_W = _B // _NW
_QROWS = 56
_NQ = _CROP // _QROWS
_AROWS = _QROWS + 8
_ACOLS = 384
_NCHUNK = _CROP // 16
_NSUB = _C * _NQ
_NA = 3
_NB = 3


def _sc_body(
    idx_hbm, t_hbm, out_hbm, idx_v, a0, a1, a2, b0, b1, b2, sem_in, sem_out
):
    wid = lax.axis_index("s") * _NC + lax.axis_index("c")
    pltpu.sync_copy(idx_hbm, idx_v)
    lanes = lax.iota(jnp.int32, 16)
    abufs = (a0, a1, a2)
    bbufs = (b0, b1, b2)

    def params(j):
        vec = idx_v[wid]
        r = jnp.sum(jnp.where(lanes == j, vec, 0))
        c = jnp.sum(jnp.where(lanes == j + _PER_W, vec, 0))
        dr = jnp.bitwise_and(r, 7)
        r0 = r - dr
        cs = pl.multiple_of(
            jnp.minimum(c - jnp.bitwise_and(c, 127), _W - _ACOLS), 128
        )
        dc = c - cs
        return r0, cs, dr, dc

    def gather_cp(pp, k, p):
        r0, cs = pp[0], pp[1]
        ch, q = divmod(k, _NQ)
        row0 = pl.multiple_of(r0 + _QROWS * q, 8)
        return pltpu.make_async_copy(
            t_hbm.at[ch, pl.ds(row0, _AROWS), pl.ds(cs, _ACOLS)],
            abufs[p],
            sem_in,
        )

    def out_cp(b_idx, k, p):
        ch, q = divmod(k, _NQ)
        return pltpu.make_async_copy(
            bbufs[p],
            out_hbm.at[b_idx, ch, pl.ds(_QROWS * q, _QROWS)],
            sem_out,
        )

    def shift(pp, pa, pb):
        dr, dc = pp[2], pp[3]
        src = abufs[pa]
        dst = bbufs[pb]
        col0 = lanes + dc

        @plsc.parallel_loop(0, _QROWS, 1, unroll=4)
        def body(i):
            row_v = jnp.full((16,), dr + i, jnp.int32)
            for kk in range(_NCHUNK):
                v = plsc.load_gather(src, [row_v, col0 + 16 * kk])
                dst[i, pl.ds(16 * kk, 16)] = v

    pp0 = params(0)
    gather_cp(pp0, 0, 0).start()
    gather_cp(pp0, 1, 1).start()

    def crop_body(j, carry):
        pp = params(j)
        b_idx = wid * _PER_W + j
        for k in range(_NSUB):
            pa = k % _NA
            pb = k % _NB
            gather_cp(pp, k, pa).wait()
            if k < _NSUB - 2:
                gather_cp(pp, k + 2, (k + 2) % _NA).start()
            else:
                @pl.when(j < _PER_W - 1)
                def _():
                    ppn = params(j + 1)
                    gather_cp(ppn, k + 2 - _NSUB, (k + 2) % _NA).start()

            if k >= _NB:
                out_cp(b_idx, k - _NB, pb).wait()
            else:

                @pl.when(j > 0)
                def _():
                    out_cp(b_idx - 1, k - _NB + _NSUB, pb).wait()

            shift(pp, pa, pb)
            out_cp(b_idx, k, pb).start()
        return carry

    lax.fori_loop(0, _PER_W, crop_body, 0)
    last = wid * _PER_W + _PER_W - 1
    for k in range(_NSUB - _NB, _NSUB):
        out_cp(last, k, k % _NB).wait()


def kernel(slice_idx, size, t):
    delta = (jnp.asarray(size) - _CROP).astype(jnp.int32)
    idx = slice_idx.astype(jnp.int32) + delta
    idx_arr = jnp.concatenate(
        [idx[:, 0].reshape(_NW, _PER_W), idx[:, 1].reshape(_NW, _PER_W)], axis=1
    )
    mesh = plsc.VectorSubcoreMesh(core_axis_name="c", subcore_axis_name="s")
    run = pl.kernel(
        _sc_body,
        out_type=jax.ShapeDtypeStruct((_B, _C, _CROP, _CROP), t.dtype),
        mesh=mesh,
        compiler_params=pltpu.CompilerParams(needs_layout_passes=False, skip_device_barrier=True),
        scratch_types=[
            pltpu.VMEM((_NW, 2 * _PER_W), jnp.int32),
            pltpu.VMEM((_AROWS, _ACOLS), jnp.float32),
            pltpu.VMEM((_AROWS, _ACOLS), jnp.float32),
            pltpu.VMEM((_AROWS, _ACOLS), jnp.float32),
            pltpu.VMEM((_QROWS, _CROP), jnp.float32),
            pltpu.VMEM((_QROWS, _CROP), jnp.float32),
            pltpu.VMEM((_QROWS, _CROP), jnp.float32),
            pltpu.SemaphoreType.DMA,
            pltpu.SemaphoreType.DMA,
        ],
    )
    return run(idx_arr, t)


# speedup vs baseline: 1.0124x; 1.0124x over previous
import jax
import jax.numpy as jnp
from jax import lax
from jax.experimental import pallas as pl
from jax.experimental.pallas import tpu as pltpu
from jax.experimental.pallas import tpu_sc as plsc

_B = 256
_CROP = 224
_C = 3
_H = 4096
_W = 4096
_NC = 2
_NS = 16
_NW = _NC * _NS
_PER_W = _B // _NW
_QROWS = 56
_NQ = _CROP // _QROWS
_AROWS = _QROWS + 8
_ACOLS = 384
_NCHUNK = _CROP // 16
_NSUB = _C * _NQ
_NA = 3
_NB = 3


def _sc_body(
    idx_hbm, t_hbm, out_hbm, idx_v, a0, a1, a2, b0, b1, b2, sem_in, sem_out
):
    wid = lax.axis_index("s") * _NC + lax.axis_index("c")
    pltpu.sync_copy(idx_hbm, idx_v)
    lanes = lax.iota(jnp.int32, 16)
    abufs = (a0, a1, a2)
    bbufs = (b0, b1, b2)

    def params(j):
        vec = idx_v[wid]
        r = jnp.sum(jnp.where(lanes == j, vec, 0))
        c = jnp.sum(jnp.where(lanes == j + _PER_W, vec, 0))
        dr = jnp.bitwise_and(r, 7)
        r0 = r - dr
        cs = pl.multiple_of(
            jnp.minimum(c - jnp.bitwise_and(c, 127), _W - _ACOLS), 128
        )
        dc = c - cs
        return r0, cs, dr, dc

    def gather_head_cp(pp, k, p):
        r0, cs = pp[0], pp[1]
        ch, q = divmod(k, _NQ)
        row0 = pl.multiple_of(r0 + _QROWS * q, 8)
        return pltpu.make_async_copy(
            t_hbm.at[ch, pl.ds(row0, _AROWS), pl.ds(cs, 256)],
            abufs[p].at[:, :256],
            sem_in,
        )

    def gather_tail_cp(pp, k, p):
        r0, cs = pp[0], pp[1]
        ch, q = divmod(k, _NQ)
        row0 = pl.multiple_of(r0 + _QROWS * q, 8)
        cs2 = pl.multiple_of(cs + 256, 128)
        return pltpu.make_async_copy(
            t_hbm.at[ch, pl.ds(row0, _AROWS), pl.ds(cs2, _ACOLS - 256)],
            abufs[p].at[:, 256:],
            sem_in,
        )

    def gather_start(pp, k, p):
        gather_head_cp(pp, k, p).start()

        @pl.when(pp[3] > 256 - _CROP)
        def _():
            gather_tail_cp(pp, k, p).start()

    def gather_wait(pp, k, p):
        gather_head_cp(pp, k, p).wait()

        @pl.when(pp[3] > 256 - _CROP)
        def _():
            gather_tail_cp(pp, k, p).wait()

    def out_cp(b_idx, k, p):
        ch, q = divmod(k, _NQ)
        return pltpu.make_async_copy(
            bbufs[p],
            out_hbm.at[b_idx, ch, pl.ds(_QROWS * q, _QROWS)],
            sem_out,
        )

    def shift(pp, pa, pb):
        dr, dc = pp[2], pp[3]
        src = abufs[pa]
        dst = bbufs[pb]
        col0 = lanes + dc

        @plsc.parallel_loop(0, _QROWS, 1, unroll=4)
        def body(i):
            row_v = jnp.full((16,), dr + i, jnp.int32)
            for kk in range(_NCHUNK):
                v = plsc.load_gather(src, [row_v, col0 + 16 * kk])
                dst[i, pl.ds(16 * kk, 16)] = v

    pp0 = params(0)
    gather_start(pp0, 0, 0)
    gather_start(pp0, 1, 1)

    def crop_body(j, carry):
        pp = params(j)
        b_idx = wid * _PER_W + j
        for k in range(_NSUB):
            pa = k % _NA
            pb = k % _NB
            gather_wait(pp, k, pa)
            if k < _NSUB - 2:
                gather_start(pp, k + 2, (k + 2) % _NA)
            else:
                @pl.when(j < _PER_W - 1)
                def _():
                    ppn = params(j + 1)
                    gather_start(ppn, k + 2 - _NSUB, (k + 2) % _NA)

            if k >= _NB:
                out_cp(b_idx, k - _NB, pb).wait()
            else:

                @pl.when(j > 0)
                def _():
                    out_cp(b_idx - 1, k - _NB + _NSUB, pb).wait()

            shift(pp, pa, pb)
            out_cp(b_idx, k, pb).start()
        return carry

    lax.fori_loop(0, _PER_W, crop_body, 0)
    last = wid * _PER_W + _PER_W - 1
    for k in range(_NSUB - _NB, _NSUB):
        out_cp(last, k, k % _NB).wait()


def kernel(slice_idx, size, t):
    delta = (jnp.asarray(size) - _CROP).astype(jnp.int32)
    idx = slice_idx.astype(jnp.int32) + delta
    idx_arr = jnp.concatenate(
        [idx[:, 0].reshape(_NW, _PER_W), idx[:, 1].reshape(_NW, _PER_W)], axis=1
    )
    mesh = plsc.VectorSubcoreMesh(core_axis_name="c", subcore_axis_name="s")
    run = pl.kernel(
        _sc_body,
        out_type=jax.ShapeDtypeStruct((_B, _C, _CROP, _CROP), t.dtype),
        mesh=mesh,
        compiler_params=pltpu.CompilerParams(needs_layout_passes=False),
        scratch_types=[
            pltpu.VMEM((_NW, 2 * _PER_W), jnp.int32),
            pltpu.VMEM((_AROWS, _ACOLS), jnp.float32),
            pltpu.VMEM((_AROWS, _ACOLS), jnp.float32),
            pltpu.VMEM((_AROWS, _ACOLS), jnp.float32),
            pltpu.VMEM((_QROWS, _CROP), jnp.float32),
            pltpu.VMEM((_QROWS, _CROP), jnp.float32),
            pltpu.VMEM((_QROWS, _CROP), jnp.float32),
            pltpu.SemaphoreType.DMA,
            pltpu.SemaphoreType.DMA,
        ],
    )
    return run(idx_arr, t)
